# baseline (speedup 1.0000x reference)
import functools

import jax
import jax.numpy as jnp
from jax import lax
from jax.experimental import pallas as pl
from jax.experimental.pallas import tpu as pltpu
from jax.experimental.pallas import tpu_sc as plsc

_VOCAB = 1000
_EMB = 32
_SBLK = 2
_H = 4


def _make_gather(n_tokens: int):
    info = plsc.get_sparse_core_info()
    nw = info.num_cores * info.num_subcores
    per_w = n_tokens // nw
    assert n_tokens % (8 * nw) == 0
    chunk = per_w // 20
    n_chunks = per_w // chunk
    assert per_w % (2 * chunk) == 0 and chunk % 8 == 0
    mesh = plsc.VectorSubcoreMesh(core_axis_name="c", subcore_axis_name="s")

    @functools.partial(
        pl.kernel,
        mesh=mesh,
        out_type=jax.ShapeDtypeStruct((n_tokens, _EMB), jnp.float32),
        scratch_types=[
            pltpu.VMEM((per_w,), jnp.int32),
            pltpu.VMEM((chunk, _EMB), jnp.float32),
            pltpu.VMEM((chunk, _EMB), jnp.float32),
            pltpu.SemaphoreType.DMA,
            pltpu.SemaphoreType.DMA,
            pltpu.SemaphoreType.DMA,
            pltpu.SemaphoreType.DMA,
        ],
        compiler_params=pltpu.CompilerParams(use_tc_tiling_on_sc=False),
    )
    def gather_k(emb_hbm, idx_hbm, out_hbm, idx_v, buf0, buf1,
                 gsem0, gsem1, wsem0, wsem1):
        wid = lax.axis_index("s") * info.num_cores + lax.axis_index("c")
        base = wid * per_w
        pltpu.sync_copy(idx_hbm.at[pl.ds(base, per_w)], idx_v)

        def start_gather(g, buf, sem):
            off = pl.multiple_of(g * chunk, 8)
            pltpu.async_copy(emb_hbm.at[idx_v.at[pl.ds(off, chunk)]],
                             buf, sem)

        def wait_gather(buf, sem):
            pltpu.make_async_copy(emb_hbm.at[pl.ds(0, chunk)], buf,
                                  sem).wait()

        def start_write(g, buf, sem):
            off = pl.multiple_of(g * chunk, 8)
            pltpu.async_copy(buf, out_hbm.at[pl.ds(base + off, chunk)], sem)

        def wait_write(buf, sem):
            pltpu.make_async_copy(buf, out_hbm.at[pl.ds(base, chunk)],
                                  sem).wait()

        start_gather(0, buf0, gsem0)
        start_gather(1, buf1, gsem1)

        def body(k, _):
            wait_gather(buf0, gsem0)
            start_write(2 * k, buf0, wsem0)
            wait_gather(buf1, gsem1)
            start_write(2 * k + 1, buf1, wsem1)

            @pl.when(k + 1 < n_chunks // 2)
            def _():
                wait_write(buf0, wsem0)
                start_gather(2 * k + 2, buf0, gsem0)
                wait_write(buf1, wsem1)
                start_gather(2 * k + 3, buf1, gsem1)
            return _

        lax.fori_loop(0, n_chunks // 2, body, None)
        wait_write(buf0, wsem0)
        wait_write(buf1, wsem1)

    return gather_k


def _proj_first(x_ref, w_ref, b_ref, out_ref):
    for s in range(_SBLK):
        for j in range(4):
            out_ref[s, :, 256 * j:256 * (j + 1)] = lax.dot_general(
                w_ref[...], x_ref[s, :, 32 * j:32 * (j + 1)],
                dimension_numbers=(((1,), (1,)), ((), ())),
                preferred_element_type=jnp.float32,
            ) + b_ref[...]


def _proj_next(buf_ref, x_ref, w_ref, b_ref, out_ref):
    del buf_ref
    _proj_first(x_ref, w_ref, b_ref, out_ref)


def kernel(idx, emb, W, b):
    bsz, seq = idx.shape
    s_per = seq // _H
    n_h = bsz * s_per
    gather = _make_gather(n_h)
    b_col = b.reshape(_VOCAB, 1)

    out_shape = jax.ShapeDtypeStruct((seq, _VOCAB, bsz), jnp.float32)
    x_spec = pl.BlockSpec((_SBLK, bsz // 4, 4 * _EMB), lambda i: (i, 0, 0))
    w_spec = pl.BlockSpec((_VOCAB, _EMB), lambda i: (0, 0))
    b_spec = pl.BlockSpec((_VOCAB, 1), lambda i: (0, 0))
    params = pltpu.CompilerParams(dimension_semantics=("arbitrary",))

    perm = (jnp.arange(bsz) % 4) * (bsz // 4) + jnp.arange(bsz) // 4
    idx_p = idx[perm, :]

    out = None
    for h in range(_H):
        flat_idx = idx_p[:, h * s_per:(h + 1) * s_per].T.reshape(n_h)
        x = gather(emb, flat_idx).reshape(s_per, bsz // 4, 4 * _EMB)

        off = h * (s_per // _SBLK)
        out_spec = pl.BlockSpec((_SBLK, _VOCAB, bsz),
                                lambda i, o=off: (o + i, 0, 0))
        if h == 0:
            out = pl.pallas_call(
                _proj_first,
                grid=(s_per // _SBLK,),
                in_specs=[x_spec, w_spec, b_spec],
                out_specs=out_spec,
                out_shape=out_shape,
                compiler_params=params,
            )(x, W, b_col)
        else:
            out = pl.pallas_call(
                _proj_next,
                grid=(s_per // _SBLK,),
                in_specs=[pl.BlockSpec(memory_space=pl.ANY),
                          x_spec, w_spec, b_spec],
                out_specs=out_spec,
                out_shape=out_shape,
                input_output_aliases={0: 0},
                compiler_params=params,
            )(out, x, W, b_col)

    return jnp.transpose(out, (2, 0, 1))


# speedup vs baseline: 1.0387x; 1.0387x over previous
import functools

import jax
import jax.numpy as jnp
from jax import lax
from jax.experimental import pallas as pl
from jax.experimental.pallas import tpu as pltpu
from jax.experimental.pallas import tpu_sc as plsc

_VOCAB = 1000
_EMB = 32
_SBLK = 4
_H = 2


def _make_gather(n_tokens: int):
    info = plsc.get_sparse_core_info()
    nw = info.num_cores * info.num_subcores
    per_w = n_tokens // nw
    assert n_tokens % (8 * nw) == 0
    chunk = per_w // 20
    n_chunks = per_w // chunk
    assert per_w % (2 * chunk) == 0 and chunk % 8 == 0
    mesh = plsc.VectorSubcoreMesh(core_axis_name="c", subcore_axis_name="s")

    @functools.partial(
        pl.kernel,
        mesh=mesh,
        out_type=jax.ShapeDtypeStruct((n_tokens, _EMB), jnp.float32),
        scratch_types=[
            pltpu.VMEM((per_w,), jnp.int32),
            pltpu.VMEM((chunk, _EMB), jnp.float32),
            pltpu.VMEM((chunk, _EMB), jnp.float32),
            pltpu.SemaphoreType.DMA,
            pltpu.SemaphoreType.DMA,
            pltpu.SemaphoreType.DMA,
            pltpu.SemaphoreType.DMA,
        ],
        compiler_params=pltpu.CompilerParams(use_tc_tiling_on_sc=False),
    )
    def gather_k(emb_hbm, idx_hbm, out_hbm, idx_v, buf0, buf1,
                 gsem0, gsem1, wsem0, wsem1):
        wid = lax.axis_index("s") * info.num_cores + lax.axis_index("c")
        base = wid * per_w
        pltpu.sync_copy(idx_hbm.at[pl.ds(base, per_w)], idx_v)

        def start_gather(g, buf, sem):
            off = pl.multiple_of(g * chunk, 8)
            pltpu.async_copy(emb_hbm.at[idx_v.at[pl.ds(off, chunk)]],
                             buf, sem)

        def wait_gather(buf, sem):
            pltpu.make_async_copy(emb_hbm.at[pl.ds(0, chunk)], buf,
                                  sem).wait()

        def start_write(g, buf, sem):
            off = pl.multiple_of(g * chunk, 8)
            pltpu.async_copy(buf, out_hbm.at[pl.ds(base + off, chunk)], sem)

        def wait_write(buf, sem):
            pltpu.make_async_copy(buf, out_hbm.at[pl.ds(base, chunk)],
                                  sem).wait()

        start_gather(0, buf0, gsem0)
        start_gather(1, buf1, gsem1)

        def body(k, _):
            wait_gather(buf0, gsem0)
            start_write(2 * k, buf0, wsem0)
            wait_gather(buf1, gsem1)
            start_write(2 * k + 1, buf1, wsem1)

            @pl.when(k + 1 < n_chunks // 2)
            def _():
                wait_write(buf0, wsem0)
                start_gather(2 * k + 2, buf0, gsem0)
                wait_write(buf1, wsem1)
                start_gather(2 * k + 3, buf1, gsem1)
            return _

        lax.fori_loop(0, n_chunks // 2, body, None)
        wait_write(buf0, wsem0)
        wait_write(buf1, wsem1)

    return gather_k


def _proj_first(x_ref, w_ref, b_ref, out_ref):
    for s in range(_SBLK):
        for j in range(4):
            out_ref[s, :, 256 * j:256 * (j + 1)] = lax.dot_general(
                w_ref[...], x_ref[s, :, 32 * j:32 * (j + 1)],
                dimension_numbers=(((1,), (1,)), ((), ())),
                preferred_element_type=jnp.float32,
            ) + b_ref[...]


def _proj_next(buf_ref, x_ref, w_ref, b_ref, out_ref):
    del buf_ref
    _proj_first(x_ref, w_ref, b_ref, out_ref)


def kernel(idx, emb, W, b):
    bsz, seq = idx.shape
    s_per = seq // _H
    n_h = bsz * s_per
    gather = _make_gather(n_h)
    b_col = b.reshape(_VOCAB, 1)

    out_shape = jax.ShapeDtypeStruct((seq, _VOCAB, bsz), jnp.float32)
    x_spec = pl.BlockSpec((_SBLK, bsz // 4, 4 * _EMB), lambda i: (i, 0, 0))
    w_spec = pl.BlockSpec((_VOCAB, _EMB), lambda i: (0, 0))
    b_spec = pl.BlockSpec((_VOCAB, 1), lambda i: (0, 0))
    params = pltpu.CompilerParams(dimension_semantics=("arbitrary",),
                                  vmem_limit_bytes=100 * 1024 * 1024)

    perm = (jnp.arange(bsz) % 4) * (bsz // 4) + jnp.arange(bsz) // 4
    idx_p = idx[perm, :]

    out = None
    for h in range(_H):
        flat_idx = idx_p[:, h * s_per:(h + 1) * s_per].T.reshape(n_h)
        x = gather(emb, flat_idx).reshape(s_per, bsz // 4, 4 * _EMB)

        off = h * (s_per // _SBLK)
        out_spec = pl.BlockSpec((_SBLK, _VOCAB, bsz),
                                lambda i, o=off: (o + i, 0, 0))
        if h == 0:
            out = pl.pallas_call(
                _proj_first,
                grid=(s_per // _SBLK,),
                in_specs=[x_spec, w_spec, b_spec],
                out_specs=out_spec,
                out_shape=out_shape,
                compiler_params=params,
            )(x, W, b_col)
        else:
            out = pl.pallas_call(
                _proj_next,
                grid=(s_per // _SBLK,),
                in_specs=[pl.BlockSpec(memory_space=pl.ANY),
                          x_spec, w_spec, b_spec],
                out_specs=out_spec,
                out_shape=out_shape,
                input_output_aliases={0: 0},
                compiler_params=params,
            )(out, x, W, b_col)

    return jnp.transpose(out, (2, 0, 1))
